# baseline (speedup 1.0000x reference)
import jax
import jax.numpy as jnp
from jax import lax
from jax.experimental import pallas as pl
from jax.experimental.pallas import tpu as pltpu
from jax.experimental.pallas import tpu_sc as plsc

N_NODES = 100000
NUM_TYPES = 128
LANES = 16
CHUNK = 160
NBUF = 4
NCHUNKS = N_NODES // CHUNK
FLAT = CHUNK * NUM_TYPES
GROUPS = CHUNK // LANES

try:
    _info = plsc.get_sparse_core_info()
    _NC = _info.num_cores
    _NW = _NC * _info.num_subcores
except Exception:
    _NC = 2
    _NW = 32
_BASE_STEPS = NCHUNKS // _NW
_EXTRA = NCHUNKS - _BASE_STEPS * _NW

_mesh = plsc.VectorSubcoreMesh(core_axis_name="c", subcore_axis_name="s")


_MAX_STEPS = _BASE_STEPS + 1


def _scratch_types():
    return (
        [pltpu.VMEM((CHUNK, NUM_TYPES), jnp.float32) for _ in range(NBUF)]
        + [pltpu.VMEM((_MAX_STEPS * CHUNK,), jnp.int32)]
        + [pltpu.SemaphoreType.DMA for _ in range(NBUF + 1)]
    )


def _onehot_body(atoms_hbm, out_hbm, *scratch):
    bufs = scratch[:NBUF]
    idxall = scratch[NBUF]
    sems = scratch[NBUF + 1 : 2 * NBUF + 1]
    sem_i = scratch[2 * NBUF + 1]

    wid = lax.axis_index("s") * _NC + lax.axis_index("c")
    lane = lax.iota(jnp.int32, LANES)
    ones = jnp.full((LANES,), 1.0, jnp.float32)
    zeros = jnp.zeros((LANES,), jnp.float32)

    def scatter(buf, step, val):
        def _s(g, carry):
            iv = idxall[pl.ds(step * CHUNK + g * LANES, LANES)]
            rows = lane + g * LANES
            plsc.store_scatter(buf, [rows, iv], val)
            return carry

        lax.fori_loop(0, GROUPS, _s, 0, unroll=5)

    def zero(buf):
        def _zero(r, carry):
            for j in range(NUM_TYPES // LANES):
                buf[r, pl.ds(j * LANES, LANES)] = zeros
            return carry

        lax.fori_loop(0, CHUNK, _zero, 0, unroll=2)

    def emit(nsteps):
        idx_dmas = []
        for i in range(nsteps):
            c = wid + i * _NW
            idx_dmas.append(
                pltpu.async_copy(
                    atoms_hbm.at[pl.ds(c * CHUNK, CHUNK)],
                    idxall.at[pl.ds(i * CHUNK, CHUNK)],
                    sem_i,
                )
            )

        pending = [None] * NBUF

        def fill(i):
            b = i % NBUF
            c = wid + i * _NW
            idx_dmas[i].wait()
            if pending[b] is not None:
                pending[b].wait()
                scatter(bufs[b], i - NBUF, zeros)
            scatter(bufs[b], i, ones)
            pending[b] = pltpu.async_copy(
                bufs[b], out_hbm.at[pl.ds(c * CHUNK, CHUNK)], sems[b]
            )

        for i in range(nsteps):
            if i < NBUF:
                zero(bufs[i])
            fill(i)
        for b in range(NBUF):
            if pending[b] is not None:
                pending[b].wait()

    @pl.when(wid < _EXTRA)
    def _():
        emit(_BASE_STEPS + 1)

    @pl.when(wid >= _EXTRA)
    def _():
        emit(_BASE_STEPS)


_onehot = pl.kernel(
    _onehot_body,
    mesh=_mesh,
    compiler_params=pltpu.CompilerParams(
        needs_layout_passes=False,
        skip_device_barrier=True,
        disable_bounds_checks=True,
        disable_semaphore_checks=True,
    ),
    out_type=jax.ShapeDtypeStruct((N_NODES, NUM_TYPES), jnp.float32),
    scratch_types=_scratch_types(),
)


def kernel(atom_types):
    return _onehot(atom_types.astype(jnp.int32))


# speedup vs baseline: 1.1027x; 1.1027x over previous
import jax
import jax.numpy as jnp
from jax import lax
from jax.experimental import pallas as pl
from jax.experimental.pallas import tpu as pltpu
from jax.experimental.pallas import tpu_sc as plsc

N_NODES = 100000
NUM_TYPES = 128
LANES = 16
CHUNK = 400
NBUF = 2
NCHUNKS = N_NODES // CHUNK
FLAT = CHUNK * NUM_TYPES
GROUPS = CHUNK // LANES

try:
    _info = plsc.get_sparse_core_info()
    _NC = _info.num_cores
    _NW = _NC * _info.num_subcores
except Exception:
    _NC = 2
    _NW = 32
_BASE_STEPS = NCHUNKS // _NW
_EXTRA = NCHUNKS - _BASE_STEPS * _NW

_mesh = plsc.VectorSubcoreMesh(core_axis_name="c", subcore_axis_name="s")


_MAX_STEPS = _BASE_STEPS + 1


def _scratch_types():
    return (
        [pltpu.VMEM((CHUNK, NUM_TYPES), jnp.float32) for _ in range(NBUF)]
        + [pltpu.VMEM((_MAX_STEPS * CHUNK,), jnp.int32)]
        + [pltpu.SemaphoreType.DMA for _ in range(NBUF + 2)]
    )


def _onehot_body(atoms_hbm, out_hbm, *scratch):
    bufs = scratch[:NBUF]
    idxall = scratch[NBUF]
    sems = scratch[NBUF + 1 : 2 * NBUF + 1]
    sem_i = scratch[2 * NBUF + 1]
    sem_h = scratch[2 * NBUF + 2]

    wid = lax.axis_index("s") * _NC + lax.axis_index("c")
    lane = lax.iota(jnp.int32, LANES)
    ones = jnp.full((LANES,), 1.0, jnp.float32)
    zeros = jnp.zeros((LANES,), jnp.float32)

    def scatter(buf, step, val, g0=0, g1=GROUPS):
        def _s(g, carry):
            iv = idxall[pl.ds(step * CHUNK + g * LANES, LANES)]
            rows = lane + g * LANES
            plsc.store_scatter(buf, [rows, iv], val)
            return carry

        lax.fori_loop(g0, g1, _s, 0, unroll=5)

    def zero(buf, r0=0, r1=CHUNK):
        def _zero(r, carry):
            for j in range(NUM_TYPES // LANES):
                buf[r, pl.ds(j * LANES, LANES)] = zeros
            return carry

        lax.fori_loop(r0, r1, _zero, 0, unroll=2)

    def emit(nsteps):
        idx_dmas = []
        for i in range(nsteps):
            c = wid + i * _NW
            idx_dmas.append(
                pltpu.async_copy(
                    atoms_hbm.at[pl.ds(c * CHUNK, CHUNK)],
                    idxall.at[pl.ds(i * CHUNK, CHUNK)],
                    sem_i,
                )
            )

        pending = [None] * NBUF
        half_dma = [None]

        def fill(i):
            b = i % NBUF
            c = wid + i * _NW
            idx_dmas[i].wait()
            if pending[b] is not None:
                if i == NBUF:
                    half_dma[0].wait()
                pending[b].wait()
                scatter(bufs[b], i - NBUF, zeros)
            scatter(bufs[b], i, ones)
            pending[b] = pltpu.async_copy(
                bufs[b], out_hbm.at[pl.ds(c * CHUNK, CHUNK)], sems[b]
            )

        HALF_G = GROUPS // 2
        HALF_R = HALF_G * LANES
        zero(bufs[0], 0, HALF_R)
        idx_dmas[0].wait()
        scatter(bufs[0], 0, ones, 0, HALF_G)
        half_dma[0] = pltpu.async_copy(
            bufs[0].at[pl.ds(0, HALF_R)],
            out_hbm.at[pl.ds(wid * CHUNK, HALF_R)],
            sem_h,
        )
        zero(bufs[0], HALF_R, CHUNK)
        scatter(bufs[0], 0, ones, HALF_G, GROUPS)
        pending[0] = pltpu.async_copy(
            bufs[0].at[pl.ds(HALF_R, CHUNK - HALF_R)],
            out_hbm.at[pl.ds(wid * CHUNK + HALF_R, CHUNK - HALF_R)],
            sems[0],
        )

        for i in range(1, nsteps):
            if i < NBUF:
                zero(bufs[i])
            fill(i)
        if nsteps <= NBUF:
            half_dma[0].wait()
        for b in range(NBUF):
            if pending[b] is not None:
                pending[b].wait()

    @pl.when(wid < _EXTRA)
    def _():
        emit(_BASE_STEPS + 1)

    @pl.when(wid >= _EXTRA)
    def _():
        emit(_BASE_STEPS)


_onehot = pl.kernel(
    _onehot_body,
    mesh=_mesh,
    compiler_params=pltpu.CompilerParams(
        needs_layout_passes=False,
        skip_device_barrier=True,
        disable_bounds_checks=True,
        disable_semaphore_checks=True,
    ),
    out_type=jax.ShapeDtypeStruct((N_NODES, NUM_TYPES), jnp.float32),
    scratch_types=_scratch_types(),
)


def kernel(atom_types):
    return _onehot(atom_types.astype(jnp.int32))


# speedup vs baseline: 1.1134x; 1.0097x over previous
import jax
import jax.numpy as jnp
from jax import lax
from jax.experimental import pallas as pl
from jax.experimental.pallas import tpu as pltpu
from jax.experimental.pallas import tpu_sc as plsc

N_NODES = 100000
NUM_TYPES = 128
LANES = 16
CHUNK = 400
NBUF = 2
NCHUNKS = N_NODES // CHUNK
FLAT = CHUNK * NUM_TYPES
GROUPS = CHUNK // LANES

try:
    _info = plsc.get_sparse_core_info()
    _NC = _info.num_cores
    _NW = _NC * _info.num_subcores
except Exception:
    _NC = 2
    _NW = 32
_BASE_STEPS = NCHUNKS // _NW
_EXTRA = NCHUNKS - _BASE_STEPS * _NW

_mesh = plsc.VectorSubcoreMesh(core_axis_name="c", subcore_axis_name="s")


_MAX_STEPS = _BASE_STEPS + 1


def _scratch_types():
    return (
        [pltpu.VMEM((CHUNK, NUM_TYPES), jnp.float32) for _ in range(NBUF)]
        + [pltpu.VMEM((_MAX_STEPS * CHUNK,), jnp.int32)]
        + [pltpu.SemaphoreType.DMA for _ in range(NBUF + 1)]
    )


def _onehot_body(atoms_hbm, out_hbm, *scratch):
    bufs = scratch[:NBUF]
    idxall = scratch[NBUF]
    sems = scratch[NBUF + 1 : 2 * NBUF + 1]
    sem_i = scratch[2 * NBUF + 1]

    wid = lax.axis_index("s") * _NC + lax.axis_index("c")
    lane = lax.iota(jnp.int32, LANES)
    ones = jnp.full((LANES,), 1.0, jnp.float32)
    zeros = jnp.zeros((LANES,), jnp.float32)

    def scatter(buf, step, val, g0=0, g1=GROUPS):
        def _s(g, carry):
            iv = idxall[pl.ds(step * CHUNK + g * LANES, LANES)]
            rows = lane + g * LANES
            plsc.store_scatter(buf, [rows, iv], val)
            return carry

        lax.fori_loop(g0, g1, _s, 0, unroll=5)

    def zero(buf, r0=0, r1=CHUNK):
        def _zero(r, carry):
            for j in range(NUM_TYPES // LANES):
                buf[r, pl.ds(j * LANES, LANES)] = zeros
            return carry

        lax.fori_loop(r0, r1, _zero, 0, unroll=2)

    def emit(nsteps):
        idx_dmas = []
        for i in range(nsteps):
            c = wid + i * _NW
            idx_dmas.append(
                pltpu.async_copy(
                    atoms_hbm.at[pl.ds(c * CHUNK, CHUNK)],
                    idxall.at[pl.ds(i * CHUNK, CHUNK)],
                    sem_i,
                )
            )

        pending = [None] * NBUF

        def fill(i):
            b = i % NBUF
            c = wid + i * _NW
            idx_dmas[i].wait()
            if pending[b] is not None:
                pending[b].wait()
                scatter(bufs[b], i - NBUF, zeros)
            scatter(bufs[b], i, ones)
            pending[b] = pltpu.async_copy(
                bufs[b], out_hbm.at[pl.ds(c * CHUNK, CHUNK)], sems[b]
            )

        for i in range(nsteps):
            if i < NBUF:
                zero(bufs[i])
            fill(i)
        for b in range(NBUF):
            if pending[b] is not None:
                pending[b].wait()

    @pl.when(wid < _EXTRA)
    def _():
        emit(_BASE_STEPS + 1)

    @pl.when(wid >= _EXTRA)
    def _():
        emit(_BASE_STEPS)


_onehot = pl.kernel(
    _onehot_body,
    mesh=_mesh,
    compiler_params=pltpu.CompilerParams(
        needs_layout_passes=False,
        skip_device_barrier=True,
        disable_bounds_checks=True,
        disable_semaphore_checks=True,
    ),
    out_type=jax.ShapeDtypeStruct((N_NODES, NUM_TYPES), jnp.float32),
    scratch_types=_scratch_types(),
)


def kernel(atom_types):
    return _onehot(atom_types.astype(jnp.int32))
